# Initial kernel scaffold; baseline (speedup 1.0000x reference)
import jax
import jax.numpy as jnp
from jax.experimental import pallas as pl


def kernel(x, target_gram):
    raise NotImplementedError("write your pallas kernel here")


# speedup vs baseline: 1.0103x; 1.0103x over previous
import functools

import jax
import jax.numpy as jnp
from jax import lax
from jax.experimental import pallas as pl
from jax.experimental.pallas import tpu as pltpu

_VMEM_LIMIT_BYTES = 48 * 1024 * 1024


def _style_loss_kernel(f_ref, tgt_ref, loss_ref, acc_ref, *, inv_norm,
                       inv_numel):
    kk = pl.program_id(0)

    @pl.when(kk == 0)
    def _():
        acc_ref[...] = jnp.zeros_like(acc_ref)

    fb = f_ref[...].astype(jnp.bfloat16)
    acc_ref[...] += lax.dot_general(
        fb, fb, (((1,), (1,)), ((), ())), preferred_element_type=jnp.float32)

    @pl.when(kk == pl.num_programs(0) - 1)
    def _():
        diff = acc_ref[...] * inv_norm - tgt_ref[...]
        loss_ref[0, 0] = jnp.sum(diff * diff) * inv_numel


def kernel(x, target_gram):
    a, b, c, d = x.shape
    m, k = a * b, c * d
    feats = x.reshape(m, k)
    tk = 2048
    nk = k // tk
    inv_norm = 1.0 / float(a * b * c * d)
    inv_numel = 1.0 / float(m * m)

    cost = pl.CostEstimate(
        flops=2 * m * m * k,
        transcendentals=0,
        bytes_accessed=m * k * 4 + m * m * 4)

    loss = pl.pallas_call(
        functools.partial(_style_loss_kernel, inv_norm=inv_norm,
                          inv_numel=inv_numel),
        out_shape=jax.ShapeDtypeStruct((1, 1), jnp.float32),
        grid_spec=pltpu.PrefetchScalarGridSpec(
            num_scalar_prefetch=0,
            grid=(nk,),
            in_specs=[
                pl.BlockSpec((m, tk), lambda kk: (0, kk)),
                pl.BlockSpec((m, m), lambda kk: (0, 0)),
            ],
            out_specs=pl.BlockSpec((1, 1), lambda kk: (0, 0),
                                   memory_space=pltpu.SMEM),
            scratch_shapes=[pltpu.VMEM((m, m), jnp.float32)],
        ),
        compiler_params=pltpu.CompilerParams(
            dimension_semantics=("arbitrary",),
            vmem_limit_bytes=_VMEM_LIMIT_BYTES),
        cost_estimate=cost,
    )(feats, target_gram)

    return x, loss[0, 0]
